# initial kernel scaffold (unmeasured)
import jax
import jax.numpy as jnp
from jax import lax
from jax.experimental import pallas as pl
from jax.experimental.pallas import tpu as pltpu


def kernel(
    x,
):
    def body(*refs):
        pass

    out_shape = jax.ShapeDtypeStruct(..., jnp.float32)
    return pl.pallas_call(body, out_shape=out_shape)(...)


# baseline (device time: 32674 ns/iter reference)
import jax
import jax.numpy as jnp
from jax import lax
from jax.experimental import pallas as pl
from jax.experimental.pallas import tpu as pltpu

N_DEV = 4


def kernel(x):
    m, n = x.shape

    def body(x_ref, out_ref, pfx_ref, sbuf_ref, send_sem, recv_sem, ack_sem):
        my = lax.axis_index("i")

        acc = x_ref[...]
        s = 1
        while s < m:
            shifted = jnp.concatenate(
                [jnp.ones((s, n), jnp.float32), acc[: m - s, :]], axis=0
            )
            acc = acc * shifted
            s *= 2

        out_ref[...] = acc

        @pl.when(my == 0)
        def _():
            pfx_ref[...] = jnp.ones((1, n), jnp.float32)

        @pl.when(my > 0)
        def _():
            recv = pltpu.make_async_remote_copy(
                src_ref=sbuf_ref,
                dst_ref=pfx_ref,
                send_sem=send_sem,
                recv_sem=recv_sem,
                device_id=(my - 1,),
                device_id_type=pl.DeviceIdType.MESH,
            )
            recv.wait_recv()

        @pl.when(my < N_DEV - 1)
        def _():
            sbuf_ref[...] = pfx_ref[...] * out_ref[pl.ds(m - 1, 1), :]
            send = pltpu.make_async_remote_copy(
                src_ref=sbuf_ref,
                dst_ref=pfx_ref,
                send_sem=send_sem,
                recv_sem=recv_sem,
                device_id=(my + 1,),
                device_id_type=pl.DeviceIdType.MESH,
            )
            send.start()
            send.wait_send()

        @pl.when(my > 0)
        def _():
            out_ref[...] = out_ref[...] * pfx_ref[...]
            pl.semaphore_signal(
                ack_sem,
                inc=1,
                device_id=(my - 1,),
                device_id_type=pl.DeviceIdType.MESH,
            )

        @pl.when(my < N_DEV - 1)
        def _():
            pl.semaphore_wait(ack_sem, 1)

    return pl.pallas_call(
        body,
        out_shape=jax.ShapeDtypeStruct((m, n), jnp.float32),
        in_specs=[pl.BlockSpec(memory_space=pltpu.VMEM)],
        out_specs=pl.BlockSpec(memory_space=pltpu.VMEM),
        scratch_shapes=[
            pltpu.VMEM((1, n), jnp.float32),
            pltpu.VMEM((1, n), jnp.float32),
            pltpu.SemaphoreType.DMA,
            pltpu.SemaphoreType.DMA,
            pltpu.SemaphoreType.REGULAR,
        ],
    )(x)
